# baseline (speedup 1.0000x reference)
import functools

import jax
import jax.numpy as jnp
from jax.experimental import pallas as pl
from jax.experimental.pallas import tpu as pltpu

_EMB = 512
_M = 1024
_UNROLL = 16


def _gather_body(idx_ref, w_ref, out_ref, sem, *, nsteps):
    k = pl.program_id(0)
    base = k * _M

    def issue(u, carry):
        m0 = u * _UNROLL
        for j in range(_UNROLL):
            row = idx_ref[base + m0 + j]
            pltpu.make_async_copy(
                w_ref.at[pl.ds(row, 1)],
                out_ref.at[pl.ds(m0 + j, 1)],
                sem,
            ).start()
        return carry

    jax.lax.fori_loop(0, _M // _UNROLL, issue, 0)
    pltpu.make_async_copy(
        w_ref.at[pl.ds(0, _M)], out_ref.at[pl.ds(0, _M)], sem
    ).wait()


def kernel(input, weight):
    bptt, bsize = input.shape
    n = bptt * bsize
    idx = input.reshape(n).astype(jnp.int32)
    w3 = weight.reshape(weight.shape[0], 1, _EMB)
    nsteps = n // _M

    grid_spec = pltpu.PrefetchScalarGridSpec(
        num_scalar_prefetch=1,
        grid=(nsteps,),
        in_specs=[pl.BlockSpec(memory_space=pl.ANY)],
        out_specs=pl.BlockSpec(
            (_M, 1, _EMB),
            lambda k, idx_ref: (k, 0, 0),
        ),
        scratch_shapes=[pltpu.SemaphoreType.DMA],
    )
    out = pl.pallas_call(
        functools.partial(_gather_body, nsteps=nsteps),
        grid_spec=grid_spec,
        out_shape=jax.ShapeDtypeStruct((n, 1, _EMB), jnp.float32),
        compiler_params=pltpu.CompilerParams(
            dimension_semantics=("arbitrary",),
            disable_bounds_checks=True,
        ),
    )(idx, w3)
    return out.reshape(bptt, bsize, _EMB)


# speedup vs baseline: 3.3944x; 3.3944x over previous
import functools

import jax
import jax.numpy as jnp
from jax.experimental import pallas as pl
from jax.experimental.pallas import tpu as pltpu

_EMB = 512
_M = 1024
_UNROLL = 16


def _gather_body(idx_ref, w_ref, out_ref, sem, *, nsteps):
    k = pl.program_id(0)
    base = k * _M

    def issue(u, carry):
        m0 = u * _UNROLL
        for j in range(_UNROLL):
            row = idx_ref[base + m0 + j]
            pltpu.make_async_copy(
                w_ref.at[pl.ds(row, 1)],
                out_ref.at[pl.ds(m0 + j, 1)],
                sem,
            ).start()
        return carry

    jax.lax.fori_loop(0, _M // _UNROLL, issue, 0)
    pltpu.make_async_copy(
        w_ref.at[pl.ds(0, _M)], out_ref.at[pl.ds(0, _M)], sem
    ).wait()


def kernel(input, weight):
    bptt, bsize = input.shape
    n = bptt * bsize
    idx = input.reshape(n).astype(jnp.int32)
    nsteps = n // _M

    grid_spec = pltpu.PrefetchScalarGridSpec(
        num_scalar_prefetch=1,
        grid=(nsteps,),
        in_specs=[pl.BlockSpec(memory_space=pl.ANY)],
        out_specs=pl.BlockSpec(
            (_M, _EMB),
            lambda k, idx_ref: (k, 0),
        ),
        scratch_shapes=[pltpu.SemaphoreType.DMA],
    )
    out = pl.pallas_call(
        functools.partial(_gather_body, nsteps=nsteps),
        grid_spec=grid_spec,
        out_shape=jax.ShapeDtypeStruct((n, _EMB), jnp.float32),
        compiler_params=pltpu.CompilerParams(
            dimension_semantics=("arbitrary",),
            disable_bounds_checks=True,
        ),
    )(idx, weight)
    return out.reshape(bptt, bsize, _EMB)
